# Initial kernel scaffold; baseline (speedup 1.0000x reference)
import jax
import jax.numpy as jnp
from jax.experimental import pallas as pl


def kernel(x_num, cat0, cat1, cat2, cat3, cat4, E0, E1, E2, E3, E4, ge0, be0, ge1, be1, ge2, be2, ge3, be3, ge4, be4, W1, c1, gf1, bf1, W2, c2, gf2, bf2, W3, c3):
    raise NotImplementedError("write your pallas kernel here")


# speedup vs baseline: 3.3396x; 3.3396x over previous
import functools

import jax
import jax.numpy as jnp
from jax import lax
from jax.experimental import pallas as pl
from jax.experimental.pallas import tpu as pltpu
from jax.experimental.pallas import tpu_sc as plsc

B = 16384
N_NUM = 13
VOCABS = [1000, 1000, 500, 200, 100]
DIMS = [500, 500, 250, 100, 50]
VOFF = [0, 1000, 2000, 2500, 2700]
DOFF = [0, 500, 1000, 1250, 1350]
VTOT = 2800
H1, H2 = 512, 256
EPS = 1e-5

NC, NS, LANES = 2, 16, 16
NW = NC * NS
BPW = B // NW
CH = 32
NCH = BPW // CH

_F32 = jnp.float32


def _gelu(z):
    return 0.5 * z * (1.0 + lax.erf(z * 0.7071067811865476))


def _tables_body(cats_ref, e0, e1, e2, e3, e4, ge_ref, be_ref, w1_ref, u_ref):
    es = [e0, e1, e2, e3, e4]
    cb = 4096
    for f in range(5):
        v, d, voff, doff = VOCABS[f], DIMS[f], VOFF[f], DOFF[f]
        iota = voff + lax.broadcasted_iota(jnp.int32, (v, 1), 0)
        cnt = jnp.zeros((v, 1), _F32)
        for c in range(B // cb):
            blk = cats_ref[f : f + 1, c * cb : (c + 1) * cb]
            cnt = cnt + jnp.sum((iota == blk).astype(_F32), axis=1, keepdims=True)
        e = es[f][...]
        m = jnp.sum(e * cnt, axis=0, keepdims=True) * (1.0 / B)
        ex2 = jnp.sum(e * e * cnt, axis=0, keepdims=True) * (1.0 / B)
        var = ex2 - m * m
        g = ge_ref[0, doff : doff + d].reshape(1, d)
        bb = be_ref[0, doff : doff + d].reshape(1, d)
        t = _gelu((e - m) * (g * lax.rsqrt(var + EPS)) + bb)
        w1_blk = w1_ref[N_NUM + doff : N_NUM + doff + d, :]
        u_ref[voff : voff + v, :] = jnp.dot(
            t, w1_blk, preferred_element_type=_F32
        )


def _tables_call(cats, e_list, ge_cat, be_cat, w1):
    return pl.pallas_call(
        _tables_body,
        out_shape=jax.ShapeDtypeStruct((VTOT, H1), _F32),
    )(cats, *e_list, ge_cat, be_cat, w1)


def _sc_gather_body(u_hbm, idx_hbm, out_hbm, i0, i1, i2, i3, i4, rows_v,
                    acc_v, sem):
    cid = lax.axis_index("c")
    sid = lax.axis_index("s")
    wid = sid * NC + cid
    base = wid * BPW
    idxs = [i0, i1, i2, i3, i4]
    for f in range(5):
        pltpu.sync_copy(idx_hbm.at[pl.ds(f * B + base, BPW)], idxs[f])

    def chunk(k, carry):
        off = k * CH
        cps = []
        for f in range(5):
            cps.append(
                pltpu.async_copy(
                    u_hbm.at[idxs[f].at[pl.ds(off, CH)]], rows_v.at[f], sem
                )
            )
        for cp in cps:
            cp.wait()

        def row(r, c2):
            for cc in range(H1 // LANES):
                sl = pl.ds(cc * LANES, LANES)
                acc_v[r, sl] = (
                    rows_v[0, r, sl]
                    + rows_v[1, r, sl]
                    + rows_v[2, r, sl]
                    + rows_v[3, r, sl]
                    + rows_v[4, r, sl]
                )
            return c2

        lax.fori_loop(0, CH, row, 0)
        pltpu.sync_copy(acc_v, out_hbm.at[pl.ds(base + off, CH)])
        return carry

    lax.fori_loop(0, NCH, chunk, 0)


def _sc_gather_call(u, idx):
    mesh = plsc.VectorSubcoreMesh(core_axis_name="c", subcore_axis_name="s")
    fn = functools.partial(
        pl.kernel,
        out_type=jax.ShapeDtypeStruct((B, H1), _F32),
        mesh=mesh,
        scratch_types=[
            pltpu.VMEM((BPW,), jnp.int32),
            pltpu.VMEM((BPW,), jnp.int32),
            pltpu.VMEM((BPW,), jnp.int32),
            pltpu.VMEM((BPW,), jnp.int32),
            pltpu.VMEM((BPW,), jnp.int32),
            pltpu.VMEM((5, CH, H1), _F32),
            pltpu.VMEM((CH, H1), _F32),
            pltpu.SemaphoreType.DMA,
        ],
    )(_sc_gather_body)
    return fn(u, idx.reshape(-1))


BM1 = 1024
NB1 = B // BM1


def _h1_of(h1s_ref, x_ref, w1n_ref, c1_ref):
    return (
        h1s_ref[...]
        + jnp.dot(x_ref[...], w1n_ref[...], preferred_element_type=_F32)
        + c1_ref[...]
    )


def _mlp1a_body(h1s_ref, x_ref, w1n_ref, c1_ref, st1_ref, s1):
    nb = pl.program_id(0)
    h1 = _h1_of(h1s_ref, x_ref, w1n_ref, c1_ref)

    @pl.when(nb == 0)
    def _():
        s1[...] = jnp.zeros_like(s1)

    s1[0:1, :] += jnp.sum(h1, axis=0, keepdims=True)
    s1[1:2, :] += jnp.sum(h1 * h1, axis=0, keepdims=True)
    st1_ref[...] = s1[...]


def _mlp1a_call(h1s, x_num, w1n, c1):
    return pl.pallas_call(
        _mlp1a_body,
        grid=(NB1,),
        in_specs=[
            pl.BlockSpec((BM1, H1), lambda nb: (nb, 0)),
            pl.BlockSpec((BM1, N_NUM), lambda nb: (nb, 0)),
            pl.BlockSpec((N_NUM, H1), lambda nb: (0, 0)),
            pl.BlockSpec((1, H1), lambda nb: (0, 0)),
        ],
        out_specs=pl.BlockSpec((2, H1), lambda nb: (0, 0)),
        out_shape=jax.ShapeDtypeStruct((2, H1), _F32),
        scratch_shapes=[pltpu.VMEM((2, H1), _F32)],
    )(h1s, x_num, w1n, c1)


def _mlp1b_body(h1s_ref, x_ref, w1n_ref, c1_ref, st1_ref, gf1_ref, bf1_ref,
                w2_ref, c2_ref, h2_ref, st2_ref, s2):
    nb = pl.program_id(0)
    h1 = _h1_of(h1s_ref, x_ref, w1n_ref, c1_ref)
    mean = st1_ref[0:1, :] * (1.0 / B)
    var = st1_ref[1:2, :] * (1.0 / B) - mean * mean
    g1 = _gelu(
        (h1 - mean) * (gf1_ref[...] * lax.rsqrt(var + EPS)) + bf1_ref[...]
    )
    h2 = jnp.dot(g1, w2_ref[...], preferred_element_type=_F32) + c2_ref[...]
    h2_ref[...] = h2

    @pl.when(nb == 0)
    def _():
        s2[...] = jnp.zeros_like(s2)

    s2[0:1, :] += jnp.sum(h2, axis=0, keepdims=True)
    s2[1:2, :] += jnp.sum(h2 * h2, axis=0, keepdims=True)
    st2_ref[...] = s2[...]


def _mlp1b_call(h1s, x_num, w1n, c1, st1, gf1, bf1, w2, c2):
    return pl.pallas_call(
        _mlp1b_body,
        grid=(NB1,),
        in_specs=[
            pl.BlockSpec((BM1, H1), lambda nb: (nb, 0)),
            pl.BlockSpec((BM1, N_NUM), lambda nb: (nb, 0)),
            pl.BlockSpec((N_NUM, H1), lambda nb: (0, 0)),
            pl.BlockSpec((1, H1), lambda nb: (0, 0)),
            pl.BlockSpec((2, H1), lambda nb: (0, 0)),
            pl.BlockSpec((1, H1), lambda nb: (0, 0)),
            pl.BlockSpec((1, H1), lambda nb: (0, 0)),
            pl.BlockSpec((H1, H2), lambda nb: (0, 0)),
            pl.BlockSpec((1, H2), lambda nb: (0, 0)),
        ],
        out_specs=[
            pl.BlockSpec((BM1, H2), lambda nb: (nb, 0)),
            pl.BlockSpec((2, H2), lambda nb: (0, 0)),
        ],
        out_shape=[
            jax.ShapeDtypeStruct((B, H2), _F32),
            jax.ShapeDtypeStruct((2, H2), _F32),
        ],
        scratch_shapes=[pltpu.VMEM((2, H2), _F32)],
    )(h1s, x_num, w1n, c1, st1, gf1, bf1, w2, c2)


BM2 = 2048
NB2 = B // BM2


def _mlp2_body(h2_ref, st2_ref, gf2_ref, bf2_ref, w3_ref, c3_ref, out_ref):
    mean = st2_ref[0:1, :] * (1.0 / B)
    var = st2_ref[1:2, :] * (1.0 / B) - mean * mean
    g2 = _gelu(
        (h2_ref[...] - mean) * (gf2_ref[...] * lax.rsqrt(var + EPS))
        + bf2_ref[...]
    )
    o = jnp.dot(g2, w3_ref[...], preferred_element_type=_F32) + c3_ref[...]
    out_ref[...] = jnp.maximum(o, 0.0)


def _mlp2_call(h2, st2, gf2, bf2, w3, c3):
    return pl.pallas_call(
        _mlp2_body,
        grid=(NB2,),
        in_specs=[
            pl.BlockSpec((BM2, H2), lambda nb: (nb, 0)),
            pl.BlockSpec((2, H2), lambda nb: (0, 0)),
            pl.BlockSpec((1, H2), lambda nb: (0, 0)),
            pl.BlockSpec((1, H2), lambda nb: (0, 0)),
            pl.BlockSpec((H2, 1), lambda nb: (0, 0)),
            pl.BlockSpec((1, 1), lambda nb: (0, 0)),
        ],
        out_specs=pl.BlockSpec((BM2, 1), lambda nb: (nb, 0)),
        out_shape=jax.ShapeDtypeStruct((B, 1), _F32),
    )(h2, st2, gf2, bf2, w3, c3)


def kernel(x_num, cat0, cat1, cat2, cat3, cat4, E0, E1, E2, E3, E4,
           ge0, be0, ge1, be1, ge2, be2, ge3, be3, ge4, be4,
           W1, c1, gf1, bf1, W2, c2, gf2, bf2, W3, c3):
    cats = jnp.stack(
        [cat0 + VOFF[0], cat1 + VOFF[1], cat2 + VOFF[2],
         cat3 + VOFF[3], cat4 + VOFF[4]]
    ).astype(jnp.int32)
    ge_cat = jnp.concatenate([ge0, ge1, ge2, ge3, ge4]).reshape(1, -1)
    be_cat = jnp.concatenate([be0, be1, be2, be3, be4]).reshape(1, -1)

    u = _tables_call(cats, [E0, E1, E2, E3, E4], ge_cat, be_cat, W1)
    h1s = _sc_gather_call(u, cats)
    w1n = W1[:N_NUM, :]
    c1r = c1.reshape(1, -1)
    st1 = _mlp1a_call(h1s, x_num, w1n, c1r)
    h2, st2 = _mlp1b_call(
        h1s, x_num, w1n, c1r, st1, gf1.reshape(1, -1),
        bf1.reshape(1, -1), W2, c2.reshape(1, -1)
    )
    return _mlp2_call(
        h2, st2, gf2.reshape(1, -1), bf2.reshape(1, -1), W3, c3.reshape(1, -1)
    )
